# baseline (speedup 1.0000x reference)
import jax
import jax.numpy as jnp
from jax.experimental import pallas as pl
from jax.experimental.pallas import tpu as pltpu

B, N, F, E, A, H, DIAMETER = 32, 64, 128, 16, 32, 128, 3
BG = 2
JC = 8

_BIG = 1e30


def _mpnn_branch(h0, e2, maskb, Wm, bm, Wu, bu):
    Wi, Wj, We = Wm[:F], Wm[F:2 * F], Wm[2 * F:]
    Wuh, Wua = Wu[:F], Wu[F:]
    eW = jnp.dot(e2, We, preferred_element_type=jnp.float32)
    eWm = eW.reshape(BG, N, N, H) + maskb[..., None]
    h = h0
    for _ in range(DIAMETER):
        ai = jnp.dot(h, Wi, preferred_element_type=jnp.float32) + bm
        aj = jnp.dot(h, Wj, preferred_element_type=jnp.float32)
        ai4 = ai.reshape(BG, N, 1, H)
        aj3 = aj.reshape(BG, N, H)
        acc = jnp.zeros((BG, N, JC, H), dtype=jnp.float32)
        for jc in range(0, N, JC):
            aj_c = aj3[:, jc:jc + JC, :].reshape(BG, 1, JC, H)
            acc = acc + jax.nn.relu(ai4 + aj_c + eWm[:, :, jc:jc + JC, :])
        agg = jnp.sum(acc, axis=2)
        h = jax.nn.relu(
            jnp.dot(h, Wuh, preferred_element_type=jnp.float32)
            + jnp.dot(agg.reshape(BG * N, H), Wua,
                      preferred_element_type=jnp.float32)
            + bu
        )
    return jnp.sum(h.reshape(BG, N, F), axis=1)


def _kernel(node_ref, e_ref, adj_ref,
            Wm_p_ref, bm_p_ref, Wu_p_ref, bu_p_ref, Wo_p_ref, bo_p_ref,
            Wm_v_ref, bm_v_ref, Wu_v_ref, bu_v_ref, Wo_v_ref, bo_v_ref,
            out_p, out_v):
    h0 = node_ref[...].reshape(BG * N, F)
    e2 = e_ref[...].reshape(BG * N * N, E)
    maskb = (adj_ref[...].astype(jnp.float32) - 1.0) * _BIG

    pooled_p = _mpnn_branch(h0, e2, maskb, Wm_p_ref[...], bm_p_ref[...],
                            Wu_p_ref[...], bu_p_ref[...])
    out_p[...] = (jnp.dot(pooled_p, Wo_p_ref[...],
                          preferred_element_type=jnp.float32)
                  + bo_p_ref[...]).reshape(1, BG, A)

    pooled_v = _mpnn_branch(h0, e2, maskb, Wm_v_ref[...], bm_v_ref[...],
                            Wu_v_ref[...], bu_v_ref[...])
    out_v[...] = (jnp.dot(pooled_v, Wo_v_ref[...],
                          preferred_element_type=jnp.float32)
                  + bo_v_ref[...]).reshape(1, BG, 1)


@jax.jit
def kernel(node_feature_mat, edge_feature_mat, adj_max,
           Wm_p, bm_p, Wu_p, bu_p, Wo_p, bo_p,
           Wm_v, bm_v, Wu_v, bu_v, Wo_v, bo_v):
    full = lambda *s: pl.BlockSpec(s, lambda i: (0,) * len(s))
    grid = B // BG

    out_p, out_v = pl.pallas_call(
        _kernel,
        grid=(grid,),
        in_specs=[
            pl.BlockSpec((BG, N, F), lambda i: (i, 0, 0)),
            pl.BlockSpec((BG, N, N, E), lambda i: (i, 0, 0, 0)),
            pl.BlockSpec((BG, N, N), lambda i: (i, 0, 0)),
            full(2 * F + E, H), full(H), full(F + H, F), full(F),
            full(F, A), full(A),
            full(2 * F + E, H), full(H), full(F + H, F), full(F),
            full(F, 1), full(1),
        ],
        out_specs=[
            pl.BlockSpec((1, BG, A), lambda i: (i, 0, 0)),
            pl.BlockSpec((1, BG, 1), lambda i: (i, 0, 0)),
        ],
        out_shape=[
            jax.ShapeDtypeStruct((B // BG, BG, A), jnp.float32),
            jax.ShapeDtypeStruct((B // BG, BG, 1), jnp.float32),
        ],
        compiler_params=pltpu.CompilerParams(
            dimension_semantics=("parallel",),
        ),
    )(node_feature_mat, edge_feature_mat, adj_max,
      Wm_p, bm_p, Wu_p, bu_p, Wo_p, bo_p,
      Wm_v, bm_v, Wu_v, bu_v, Wo_v, bo_v)

    return out_p.reshape(B, A), out_v.reshape(-1)


# speedup vs baseline: 1.1133x; 1.1133x over previous
import jax
import jax.numpy as jnp
from jax.experimental import pallas as pl
from jax.experimental.pallas import tpu as pltpu

B, N, F, E, A, H, DIAMETER = 32, 64, 128, 16, 32, 128, 3
BG = 2
JC = 8

_BIG = 1e30


def _mpnn_branch(h0, e2, maskb, Wm, bm, Wu, bu):
    Wi, Wj, We = Wm[:F], Wm[F:2 * F], Wm[2 * F:]
    Wuh, Wua = Wu[:F], Wu[F:]
    eW = jnp.dot(e2, We, preferred_element_type=jnp.float32)
    eWm = eW.reshape(BG, N, N, H) + maskb[..., None]
    h = h0
    for _ in range(DIAMETER):
        ai = jnp.dot(h, Wi, preferred_element_type=jnp.float32) + bm
        aj = jnp.dot(h, Wj, preferred_element_type=jnp.float32)
        ai4 = ai.reshape(BG, N, 1, H)
        aj3 = aj.reshape(BG, N, H)
        acc = jnp.zeros((BG, N, JC, H), dtype=jnp.float32)
        for jc in range(0, N, JC):
            aj_c = aj3[:, jc:jc + JC, :].reshape(BG, 1, JC, H)
            acc = acc + jax.nn.relu(ai4 + aj_c + eWm[:, :, jc:jc + JC, :])
        agg = jnp.sum(acc, axis=2)
        h = jax.nn.relu(
            jnp.dot(h, Wuh, preferred_element_type=jnp.float32)
            + jnp.dot(agg.reshape(BG * N, H), Wua,
                      preferred_element_type=jnp.float32)
            + bu
        )
    return jnp.sum(h.reshape(BG, N, F), axis=1)


def _kernel(node_ref, e_ref, adj_ref,
            Wm_p_ref, bm_p_ref, Wu_p_ref, bu_p_ref, Wo_p_ref, bo_p_ref,
            Wm_v_ref, bm_v_ref, Wu_v_ref, bu_v_ref, Wo_v_ref, bo_v_ref,
            out_p, out_v):
    h0 = node_ref[...].reshape(BG * N, F)
    e2 = e_ref[...]
    maskb = (adj_ref[...].astype(jnp.float32) - 1.0) * _BIG

    pooled_p = _mpnn_branch(h0, e2, maskb, Wm_p_ref[...], bm_p_ref[...],
                            Wu_p_ref[...], bu_p_ref[...])
    out_p[...] = (jnp.dot(pooled_p, Wo_p_ref[...],
                          preferred_element_type=jnp.float32)
                  + bo_p_ref[...]).reshape(1, BG, A)

    pooled_v = _mpnn_branch(h0, e2, maskb, Wm_v_ref[...], bm_v_ref[...],
                            Wu_v_ref[...], bu_v_ref[...])
    out_v[...] = (jnp.dot(pooled_v, Wo_v_ref[...],
                          preferred_element_type=jnp.float32)
                  + bo_v_ref[...]).reshape(1, BG, 1)


@jax.jit
def kernel(node_feature_mat, edge_feature_mat, adj_max,
           Wm_p, bm_p, Wu_p, bu_p, Wo_p, bo_p,
           Wm_v, bm_v, Wu_v, bu_v, Wo_v, bo_v):
    full = lambda *s: pl.BlockSpec(s, lambda i: (0,) * len(s))
    grid = B // BG

    out_p, out_v = pl.pallas_call(
        _kernel,
        grid=(grid,),
        in_specs=[
            pl.BlockSpec((BG, N, F), lambda i: (i, 0, 0)),
            pl.BlockSpec((BG * N * N, E), lambda i: (i, 0)),
            pl.BlockSpec((BG, N, N), lambda i: (i, 0, 0)),
            full(2 * F + E, H), full(H), full(F + H, F), full(F),
            full(F, A), full(A),
            full(2 * F + E, H), full(H), full(F + H, F), full(F),
            full(F, 1), full(1),
        ],
        out_specs=[
            pl.BlockSpec((1, BG, A), lambda i: (i, 0, 0)),
            pl.BlockSpec((1, BG, 1), lambda i: (i, 0, 0)),
        ],
        out_shape=[
            jax.ShapeDtypeStruct((B // BG, BG, A), jnp.float32),
            jax.ShapeDtypeStruct((B // BG, BG, 1), jnp.float32),
        ],
        compiler_params=pltpu.CompilerParams(
            dimension_semantics=("parallel",),
        ),
    )(node_feature_mat, edge_feature_mat.reshape(B * N * N, E), adj_max,
      Wm_p, bm_p, Wu_p, bu_p, Wo_p, bo_p,
      Wm_v, bm_v, Wu_v, bu_v, Wo_v, bo_v)

    return out_p.reshape(B, A), out_v.reshape(-1)


# speedup vs baseline: 1.1959x; 1.0743x over previous
import jax
import jax.numpy as jnp
from jax.experimental import pallas as pl
from jax.experimental.pallas import tpu as pltpu

B, N, F, E, A, H, DIAMETER = 32, 64, 128, 16, 32, 128, 3
BG = 4
JC = 8

_BIG = 1e30


def _mpnn_branch(h0, e2, maskb, Wm, bm, Wu, bu):
    Wi, Wj, We = Wm[:F], Wm[F:2 * F], Wm[2 * F:]
    Wuh, Wua = Wu[:F], Wu[F:]
    eW = jnp.dot(e2, We, preferred_element_type=jnp.float32)
    eWm = eW.reshape(BG, N, N, H) + maskb[..., None]
    h = h0
    for _ in range(DIAMETER):
        ai = jnp.dot(h, Wi, preferred_element_type=jnp.float32) + bm
        aj = jnp.dot(h, Wj, preferred_element_type=jnp.float32)
        ai4 = ai.reshape(BG, N, 1, H)
        aj3 = aj.reshape(BG, N, H)
        acc = jnp.zeros((BG, N, JC, H), dtype=jnp.float32)
        for jc in range(0, N, JC):
            aj_c = aj3[:, jc:jc + JC, :].reshape(BG, 1, JC, H)
            acc = acc + jax.nn.relu(ai4 + aj_c + eWm[:, :, jc:jc + JC, :])
        agg = jnp.sum(acc, axis=2)
        h = jax.nn.relu(
            jnp.dot(h, Wuh, preferred_element_type=jnp.float32)
            + jnp.dot(agg.reshape(BG * N, H), Wua,
                      preferred_element_type=jnp.float32)
            + bu
        )
    return jnp.sum(h.reshape(BG, N, F), axis=1)


def _kernel(node_ref, e_ref, adj_ref,
            Wm_p_ref, bm_p_ref, Wu_p_ref, bu_p_ref, Wo_p_ref, bo_p_ref,
            Wm_v_ref, bm_v_ref, Wu_v_ref, bu_v_ref, Wo_v_ref, bo_v_ref,
            out_p, out_v):
    h0 = node_ref[...].reshape(BG * N, F)
    e2 = e_ref[...]
    maskb = (adj_ref[...].astype(jnp.float32) - 1.0) * _BIG

    pooled_p = _mpnn_branch(h0, e2, maskb, Wm_p_ref[...], bm_p_ref[...],
                            Wu_p_ref[...], bu_p_ref[...])
    out_p[...] = (jnp.dot(pooled_p, Wo_p_ref[...],
                          preferred_element_type=jnp.float32)
                  + bo_p_ref[...]).reshape(1, BG, A)

    pooled_v = _mpnn_branch(h0, e2, maskb, Wm_v_ref[...], bm_v_ref[...],
                            Wu_v_ref[...], bu_v_ref[...])
    out_v[...] = (jnp.dot(pooled_v, Wo_v_ref[...],
                          preferred_element_type=jnp.float32)
                  + bo_v_ref[...]).reshape(1, BG, 1)


@jax.jit
def kernel(node_feature_mat, edge_feature_mat, adj_max,
           Wm_p, bm_p, Wu_p, bu_p, Wo_p, bo_p,
           Wm_v, bm_v, Wu_v, bu_v, Wo_v, bo_v):
    full = lambda *s: pl.BlockSpec(s, lambda i: (0,) * len(s))
    grid = B // BG

    out_p, out_v = pl.pallas_call(
        _kernel,
        grid=(grid,),
        in_specs=[
            pl.BlockSpec((BG, N, F), lambda i: (i, 0, 0)),
            pl.BlockSpec((BG * N * N, E), lambda i: (i, 0)),
            pl.BlockSpec((BG, N, N), lambda i: (i, 0, 0)),
            full(2 * F + E, H), full(H), full(F + H, F), full(F),
            full(F, A), full(A),
            full(2 * F + E, H), full(H), full(F + H, F), full(F),
            full(F, 1), full(1),
        ],
        out_specs=[
            pl.BlockSpec((1, BG, A), lambda i: (i, 0, 0)),
            pl.BlockSpec((1, BG, 1), lambda i: (i, 0, 0)),
        ],
        out_shape=[
            jax.ShapeDtypeStruct((B // BG, BG, A), jnp.float32),
            jax.ShapeDtypeStruct((B // BG, BG, 1), jnp.float32),
        ],
        compiler_params=pltpu.CompilerParams(
            dimension_semantics=("parallel",),
        ),
    )(node_feature_mat, edge_feature_mat.reshape(B * N * N, E), adj_max,
      Wm_p, bm_p, Wu_p, bu_p, Wo_p, bo_p,
      Wm_v, bm_v, Wu_v, bu_v, Wo_v, bo_v)

    return out_p.reshape(B, A), out_v.reshape(-1)
